# baseline (speedup 1.0000x reference)
import functools
from math import ceil

import jax
import jax.numpy as jnp
from jax import lax
from jax.experimental import pallas as pl
from jax.experimental.pallas import tpu as pltpu, tpu_sc as plsc

ALPHA = 0.9
K_SEL = 40
N_ROWS = 128
V = 100000
K_BIG = int(ceil((1.0 - ALPHA) * V))

NC, NS, L = 2, 16, 16
NW = NC * NS
ROWS_PER_W = N_ROWS // NW

NVEC = V // L
U = 5
NGRP = NVEC // U
HBITS = 11
NBINS1 = 1 << HBITS
SHIFT1 = 32 - HBITS
NBINS2 = 2048
NBINS3 = 1024
CAP40 = 512

_MIN32 = -0x80000000
_M31 = 0x7FFFFFFF


def _ordmap_f(v):
    b = lax.bitcast_convert_type(v, jnp.int32)
    return b ^ ((b >> 31) & _M31)


def _unmap_f(s):
    i = s ^ ((s >> 31) & _M31)
    return lax.bitcast_convert_type(i, jnp.float32)


def _ub(s):
    return s ^ _MIN32


def _scalar(splat):
    iota = lax.iota(jnp.int32, L)
    return jnp.sum(jnp.where(iota == 0, splat, 0))


def _poly_ln(x):
    b = lax.bitcast_convert_type(x, jnp.int32)
    e = ((b >> 23) & jnp.int32(0xFF)) - jnp.int32(127)
    m = lax.bitcast_convert_type((b & jnp.int32(0x7FFFFF)) | jnp.int32(0x3F800000),
                                 jnp.float32)
    big = m > jnp.float32(1.4142135381698608)
    m = jnp.where(big, m * jnp.float32(0.5), m)
    e = e + big.astype(jnp.int32)
    s = (m - jnp.float32(1.0)) / (m + jnp.float32(1.0))
    s2 = s * s
    t = jnp.float32(1.0 / 7.0) + s2 * jnp.float32(1.0 / 9.0)
    t = jnp.float32(1.0 / 5.0) + s2 * t
    t = jnp.float32(1.0 / 3.0) + s2 * t
    t = jnp.float32(1.0) + s2 * t
    t = jnp.float32(2.0) * s * t
    return e.astype(jnp.float32) * jnp.float32(0.6931471805599453) + t


def _zero_i32(ref, n):
    z = jnp.zeros((L,), jnp.int32)

    def body(i, c):
        for t in range(4):
            ref[pl.ds((4 * i + t) * L, L)] = z
        return c

    lax.fori_loop(0, n // (4 * L), body, 0)


def _zero_f32(ref, n):
    z = jnp.zeros((L,), jnp.float32)

    def body(i, c):
        for t in range(4):
            ref[pl.ds((4 * i + t) * L, L)] = z
        return c

    lax.fori_loop(0, n // (4 * L), body, 0)


def _scan_hist(hist_ref, nbins, k):
    iota = lax.iota(jnp.int32, L)
    nv = nbins // L

    def pha(i, carry):
        acc, jstar, astar, found = carry
        j = nv - 1 - i
        h = hist_ref[pl.ds(j * L, L)]
        tot = jnp.sum(h)
        acc2 = acc + tot
        upd = (acc2 >= k) & (found == 0)
        jstar = jnp.where(upd, j, jstar)
        astar = jnp.where(upd, acc, astar)
        found = jnp.where(upd, 1, found)
        return acc2, jstar, astar, found

    _, jstar, astar, _ = lax.fori_loop(
        0, nv, pha,
        (jnp.int32(0), jnp.int32(0), jnp.int32(0), jnp.int32(0)))

    h = hist_ref[pl.ds(jstar * L, L)]
    hr = lax.rev(h, (0,))
    cs = plsc.cumsum(hr)
    tot = astar + cs
    hit = tot >= k
    ffs = plsc.all_reduce_ffs(hit)
    sel = iota == ffs
    cs_at = jnp.sum(jnp.where(sel, cs, 0))
    h_at = jnp.sum(jnp.where(sel, hr, 0))
    B = jstar * L + 15 - _scalar(ffs)
    gt = astar + cs_at - h_at
    return B, gt


def _count_ge_arr(src_ref, n, thr_s):
    iota = lax.iota(jnp.int32, L)
    nv = (n + L - 1) // L

    def body(i, cnt):
        s = src_ref[pl.ds(i * L, L)]
        valid = (i * L + iota) < n
        m = valid & (s >= thr_s)
        return cnt + plsc.all_reduce_population_count(m)

    cnt = lax.fori_loop(0, nv, body, jnp.zeros((L,), jnp.int32))
    return _scalar(cnt)


def _kth_largest_arr(src_ref, n, k):
    cnt0 = _count_ge_arr(src_ref, n, jnp.int32(0))
    T = jnp.where(cnt0 >= k, jnp.int32(0), _MIN32)

    def body(it, T):
        cand = T | (jnp.int32(1) << (30 - it))
        cnt = _count_ge_arr(src_ref, n, cand)
        return jnp.where(cnt >= k, cand, T)

    return lax.fori_loop(0, 31, body, T)


def _process_tensor(row_v, h1_v, h2_v, h3_v, e3_v, collect40, c40v_v, c40i_v):
    iota = lax.iota(jnp.int32, L)
    ones = jnp.ones((L,), jnp.int32)

    _zero_i32(h1_v, NBINS1)

    def p1(i, c):
        base = i * (U * L)
        for t in range(U):
            v = row_v[pl.ds(base + t * L, L)]
            s = _ordmap_f(v)
            b = lax.shift_right_logical(_ub(s), SHIFT1)
            plsc.addupdate_scatter(h1_v, [b], ones)
        return c

    lax.fori_loop(0, NGRP, p1, 0)

    B1, gt1 = _scan_hist(h1_v, NBINS1, jnp.int32(K_BIG))
    if collect40:
        B140, _ = _scan_hist(h1_v, NBINS1, jnp.int32(K_SEL))
    tlo = _unmap_f(lax.shift_left(B1, SHIFT1) ^ _MIN32)

    _zero_i32(h2_v, NBINS2)

    def p2(i, carry):
        b40, acc = carry
        base = i * (U * L)
        vs, ss, bs = [], [], []
        for t in range(U):
            v = row_v[pl.ds(base + t * L, L)]
            s = _ordmap_f(v)
            b = lax.shift_right_logical(_ub(s), SHIFT1)
            vs.append(v); ss.append(s); bs.append(b)
            e = jnp.exp(v - tlo)
            acc = acc + jnp.where(b > B1, e, jnp.float32(0.0))
            b2 = lax.shift_right_logical(_ub(s), 10) & jnp.int32(NBINS2 - 1)
            plsc.addupdate_scatter(h2_v, [b2], ones, mask=b == B1)
        if collect40:
            m40s = [b >= B140 for b in bs]
            anym = m40s[0]
            for t in range(1, U):
                anym = anym | m40s[t]
            hit = _scalar(plsc.all_reduce_population_count(anym)) > 0

            def collect(b40):
                bb = b40
                for t in range(U):
                    m = m40s[t]
                    offs = bb + plsc.cumsum(m.astype(jnp.int32)) - 1
                    ok = m & (offs < CAP40)
                    plsc.store_scatter(c40v_v, [offs], ss[t], mask=ok)
                    plsc.store_scatter(c40i_v, [offs],
                                       base + t * L + iota, mask=ok)
                    bb = bb + plsc.all_reduce_population_count(m)
                return bb

            b40 = lax.cond(hit, collect, lambda b: b, b40)
        return b40, acc

    b40, acc = lax.fori_loop(
        0, NGRP, p2,
        (jnp.zeros((L,), jnp.int32), jnp.zeros((L,), jnp.float32)))
    acc_hi = jnp.sum(acc)

    k2 = jnp.int32(K_BIG) - gt1
    B2, gt2 = _scan_hist(h2_v, NBINS2, k2)

    _zero_i32(h3_v, NBINS3)
    _zero_f32(e3_v, NBINS3)

    def p3(i, carry):
        zmid = carry
        base = i * (U * L)
        for t in range(U):
            v = row_v[pl.ds(base + t * L, L)]
            s = _ordmap_f(v)
            b1 = lax.shift_right_logical(_ub(s), SHIFT1)
            b2 = lax.shift_right_logical(_ub(s), 10) & jnp.int32(NBINS2 - 1)
            inb1 = b1 == B1
            e = jnp.exp(v - tlo)
            zmid = zmid + jnp.where(inb1 & (b2 > B2), e, jnp.float32(0.0))
            m = inb1 & (b2 == B2)
            b3 = _ub(s) & jnp.int32(NBINS3 - 1)
            plsc.addupdate_scatter(h3_v, [b3], ones, mask=m)
            plsc.addupdate_scatter(e3_v, [b3], e, mask=m)
        return zmid

    zmid = lax.fori_loop(0, NGRP, p3, jnp.zeros((L,), jnp.float32))

    k3 = k2 - gt2
    B3, _ = _scan_hist(h3_v, NBINS3, k3)

    def tailp(i, carry):
        csum, zsum = carry
        h3 = h3_v[pl.ds(i * L, L)]
        e3 = e3_v[pl.ds(i * L, L)]
        m = (i * L + iota) >= B3
        csum = csum + jnp.where(m, h3, 0)
        zsum = zsum + jnp.where(m, e3, jnp.float32(0.0))
        return csum, zsum

    csum, zsum = lax.fori_loop(
        0, NBINS3 // L, tailp,
        (jnp.zeros((L,), jnp.int32), jnp.zeros((L,), jnp.float32)))

    t_ub = lax.shift_left(B1, SHIFT1) | lax.shift_left(B2, 10) | B3
    t_s = t_ub ^ _MIN32
    t_f = _unmap_f(t_s)
    cge = gt1 + gt2 + jnp.sum(csum)
    corr = jnp.sum(jnp.where(iota == 0,
                             jnp.exp(jnp.full((L,), t_f - tlo, jnp.float32)),
                             jnp.float32(0.0)))
    z = acc_hi + jnp.sum(zmid) + jnp.sum(zsum) \
        - (cge - jnp.int32(K_BIG)).astype(jnp.float32) * corr

    if collect40:
        n40 = jnp.minimum(_scalar(b40), CAP40)
        return t_s, tlo, z, n40
    return t_s, tlo, z, jnp.int32(0)


def _sc_body(exp_hbm, ama_hbm, out_hbm, row_v, h1_v, h2_v, h3_v, e3_v,
             c40v_v, c40i_v, sel_s_v, sel_i_v, score_v):
    iota = lax.iota(jnp.int32, L)

    def row_body(r, carry):
        row = wid = lax.axis_index("s") * NC + lax.axis_index("c")
        row = wid * ROWS_PER_W + r

        pltpu.sync_copy(exp_hbm.at[row], row_v)
        te_s, tlo_e, z_e, n40 = _process_tensor(
            row_v, h1_v, h2_v, h3_v, e3_v, True, c40v_v, c40i_v)

        T40 = _kth_largest_arr(c40v_v, n40, jnp.int32(K_SEL))
        cnt_gt = _count_ge_arr(c40v_v, n40, T40 + 1)
        need = jnp.int32(K_SEL) - cnt_gt
        nv40 = (n40 + L - 1) // L

        for j in range(3):
            sel_s_v[pl.ds(j * L, L)] = jnp.zeros((L,), jnp.int32)
            sel_i_v[pl.ds(j * L, L)] = jnp.zeros((L,), jnp.int32)

        def selbody(i, carry):
            kb, tb = carry
            s = c40v_v[pl.ds(i * L, L)]
            ix = c40i_v[pl.ds(i * L, L)]
            valid = (i * L + iota) < n40
            mgt = valid & (s > T40)
            mtie = valid & (s == T40)
            trank = tb + plsc.cumsum(mtie.astype(jnp.int32)) - 1
            keep = mgt | (mtie & (trank < need))
            offs = kb + plsc.cumsum(keep.astype(jnp.int32)) - 1
            okm = keep & (offs < jnp.int32(3 * L))
            plsc.store_scatter(sel_s_v, [offs], s, mask=okm)
            plsc.store_scatter(sel_i_v, [offs], ix, mask=okm)
            kb = kb + plsc.all_reduce_population_count(keep)
            tb = tb + plsc.all_reduce_population_count(mtie)
            return kb, tb

        z0 = jnp.zeros((L,), jnp.int32)
        lax.fori_loop(0, nv40, selbody, (z0, z0))

        pltpu.sync_copy(ama_hbm.at[row], row_v)
        ta_s, tlo_a, z_a, _ = _process_tensor(
            row_v, h1_v, h2_v, h3_v, e3_v, False, c40v_v, c40i_v)

        for j in range(3):
            s40 = sel_s_v[pl.ds(j * L, L)]
            i40 = sel_i_v[pl.ds(j * L, L)]
            v40 = _unmap_f(s40)
            pe = jnp.exp(v40 - tlo_e) / z_e
            la = plsc.load_gather(row_v, [i40])
            sa = _ordmap_f(la)
            pa = jnp.where(sa >= ta_s, jnp.exp(la - tlo_a),
                           jnp.float32(0.0)) / z_a
            ratio = pe / (pa + jnp.float32(1e-8))
            score_v[pl.ds(j * L, L)] = _poly_ln(ratio)

        ninf = jnp.full((L,), -jnp.inf, jnp.float32)

        def memset(i, c):
            base = i * (U * L)
            for t in range(U):
                row_v[pl.ds(base + t * L, L)] = ninf
            return c

        lax.fori_loop(0, NGRP, memset, 0)
        for j in range(3):
            i40 = sel_i_v[pl.ds(j * L, L)]
            sc = score_v[pl.ds(j * L, L)]
            slot = j * L + iota
            plsc.store_scatter(row_v, [i40], sc, mask=slot < jnp.int32(K_SEL))
        pltpu.sync_copy(row_v, out_hbm.at[row])
        return carry

    lax.fori_loop(0, ROWS_PER_W, row_body, 0)


def _make_sc_kernel():
    mesh = plsc.VectorSubcoreMesh(core_axis_name="c", subcore_axis_name="s")
    return pl.kernel(
        _sc_body,
        out_type=[jax.ShapeDtypeStruct((N_ROWS, V), jnp.float32)],
        mesh=mesh,
        scratch_types=[
            pltpu.VMEM((V,), jnp.float32),
            pltpu.VMEM((NBINS1,), jnp.int32),
            pltpu.VMEM((NBINS2,), jnp.int32),
            pltpu.VMEM((NBINS3,), jnp.int32),
            pltpu.VMEM((NBINS3,), jnp.float32),
            pltpu.VMEM((CAP40,), jnp.int32),
            pltpu.VMEM((CAP40,), jnp.int32),
            pltpu.VMEM((3 * L,), jnp.int32),
            pltpu.VMEM((3 * L,), jnp.int32),
            pltpu.VMEM((3 * L,), jnp.float32),
        ],
        compiler_params=pltpu.CompilerParams(needs_layout_passes=False),
    )


def kernel(logits_exp, logits_ama):
    (out,) = _make_sc_kernel()(logits_exp, logits_ama)
    return out


# speedup vs baseline: 2.7487x; 2.7487x over previous
import functools
from math import ceil

import jax
import jax.numpy as jnp
from jax import lax
from jax.experimental import pallas as pl
from jax.experimental.pallas import tpu as pltpu, tpu_sc as plsc

ALPHA = 0.9
K_SEL = 40
N_ROWS = 128
V = 100000
K_BIG = int(ceil((1.0 - ALPHA) * V))

NC, NS, L = 2, 16, 16
NW = NC * NS
ROWS_PER_W = N_ROWS // NW

NVEC = V // L
U = 5
NGRP = NVEC // U
HBITS = 11
NBINS1 = 1 << HBITS
SHIFT1 = 32 - HBITS
NBINS2 = 2048
NBINS3 = 1024
CAP40 = 512

_MIN32 = -0x80000000
_M31 = 0x7FFFFFFF


def _ordmap_f(v):
    b = lax.bitcast_convert_type(v, jnp.int32)
    return b ^ ((b >> 31) & _M31)


def _unmap_f(s):
    i = s ^ ((s >> 31) & _M31)
    return lax.bitcast_convert_type(i, jnp.float32)


def _ub(s):
    return s ^ _MIN32


def _scalar(splat):
    iota = lax.iota(jnp.int32, L)
    return jnp.sum(jnp.where(iota == 0, splat, 0))


def _poly_ln(x):
    b = lax.bitcast_convert_type(x, jnp.int32)
    e = ((b >> 23) & jnp.int32(0xFF)) - jnp.int32(127)
    m = lax.bitcast_convert_type((b & jnp.int32(0x7FFFFF)) | jnp.int32(0x3F800000),
                                 jnp.float32)
    big = m > jnp.float32(1.4142135381698608)
    m = jnp.where(big, m * jnp.float32(0.5), m)
    e = e + big.astype(jnp.int32)
    s = (m - jnp.float32(1.0)) / (m + jnp.float32(1.0))
    s2 = s * s
    t = jnp.float32(1.0 / 7.0) + s2 * jnp.float32(1.0 / 9.0)
    t = jnp.float32(1.0 / 5.0) + s2 * t
    t = jnp.float32(1.0 / 3.0) + s2 * t
    t = jnp.float32(1.0) + s2 * t
    t = jnp.float32(2.0) * s * t
    return e.astype(jnp.float32) * jnp.float32(0.6931471805599453) + t


def _zero_i32(ref, n):
    z = jnp.zeros((L,), jnp.int32)

    @plsc.parallel_loop(0, n // L, unroll=8)
    def body(i):
        ref[pl.ds(i * L, L)] = z


def _zero_f32(ref, n):
    z = jnp.zeros((L,), jnp.float32)

    @plsc.parallel_loop(0, n // L, unroll=8)
    def body(i):
        ref[pl.ds(i * L, L)] = z


def _scan_hist(hist_ref, nbins, k):
    iota = lax.iota(jnp.int32, L)
    nv = nbins // L

    def pha(i, carry):
        acc, jstar, astar, found = carry
        j = nv - 1 - i
        h = hist_ref[pl.ds(j * L, L)]
        tot = jnp.sum(h)
        acc2 = acc + tot
        upd = (acc2 >= k) & (found == 0)
        jstar = jnp.where(upd, j, jstar)
        astar = jnp.where(upd, acc, astar)
        found = jnp.where(upd, 1, found)
        return acc2, jstar, astar, found

    _, jstar, astar, _ = lax.fori_loop(
        0, nv, pha,
        (jnp.int32(0), jnp.int32(0), jnp.int32(0), jnp.int32(0)))

    h = hist_ref[pl.ds(jstar * L, L)]
    hr = lax.rev(h, (0,))
    cs = plsc.cumsum(hr)
    tot = astar + cs
    hit = tot >= k
    ffs = plsc.all_reduce_ffs(hit)
    sel = iota == ffs
    cs_at = jnp.sum(jnp.where(sel, cs, 0))
    h_at = jnp.sum(jnp.where(sel, hr, 0))
    B = jstar * L + 15 - _scalar(ffs)
    gt = astar + cs_at - h_at
    return B, gt


def _count_ge_arr(src_ref, n, thr_s):
    iota = lax.iota(jnp.int32, L)
    nv = (n + L - 1) // L

    def body(i, cnt):
        s = src_ref[pl.ds(i * L, L)]
        valid = (i * L + iota) < n
        m = valid & (s >= thr_s)
        return cnt + plsc.all_reduce_population_count(m)

    cnt = lax.fori_loop(0, nv, body, jnp.zeros((L,), jnp.int32))
    return _scalar(cnt)


def _kth_largest_arr(src_ref, n, k):
    cnt0 = _count_ge_arr(src_ref, n, jnp.int32(0))
    T = jnp.where(cnt0 >= k, jnp.int32(0), _MIN32)

    def body(it, T):
        cand = T | (jnp.int32(1) << (30 - it))
        cnt = _count_ge_arr(src_ref, n, cand)
        return jnp.where(cnt >= k, cand, T)

    return lax.fori_loop(0, 31, body, T)


def _process_tensor(row_v, h1_v, h2_v, h3_v, e3_v, collect40, c40v_v, c40i_v):
    iota = lax.iota(jnp.int32, L)
    ones = jnp.ones((L,), jnp.int32)

    _zero_i32(h1_v, NBINS1)

    @plsc.parallel_loop(0, NVEC, unroll=10)
    def p1(i):
        v = row_v[pl.ds(i * L, L)]
        s = _ordmap_f(v)
        b = lax.shift_right_logical(_ub(s), SHIFT1)
        plsc.addupdate_scatter(h1_v, [b], ones)

    B1, gt1 = _scan_hist(h1_v, NBINS1, jnp.int32(K_BIG))
    if collect40:
        B140, _ = _scan_hist(h1_v, NBINS1, jnp.int32(K_SEL))
    tlo = _unmap_f(lax.shift_left(B1, SHIFT1) ^ _MIN32)

    _zero_i32(h2_v, NBINS2)

    p2_carry = (jnp.zeros((L,), jnp.int32), jnp.zeros((L,), jnp.float32))

    @plsc.parallel_loop(0, NVEC, unroll=5, carry=p2_carry)
    def p2(i, carry):
        b40, acc = carry
        v = row_v[pl.ds(i * L, L)]
        s = _ordmap_f(v)
        b = lax.shift_right_logical(_ub(s), SHIFT1)
        e = jnp.exp(v - tlo)
        acc = acc + jnp.where(b > B1, e, jnp.float32(0.0))
        b2 = lax.shift_right_logical(_ub(s), 10) & jnp.int32(NBINS2 - 1)
        plsc.addupdate_scatter(h2_v, [b2], ones, mask=b == B1)
        if collect40:
            m40 = b >= B140
            hit = _scalar(plsc.all_reduce_population_count(m40)) > 0

            def collect(bb):
                offs = bb + plsc.cumsum(m40.astype(jnp.int32)) - 1
                ok = m40 & (offs < CAP40)
                plsc.store_scatter(c40v_v, [offs], s, mask=ok)
                plsc.store_scatter(c40i_v, [offs], i * L + iota, mask=ok)
                return bb + plsc.all_reduce_population_count(m40)

            b40 = lax.cond(hit, collect, lambda bb: bb, b40)
        return b40, acc

    b40, acc = p2
    acc_hi = jnp.sum(acc)

    k2 = jnp.int32(K_BIG) - gt1
    B2, gt2 = _scan_hist(h2_v, NBINS2, k2)

    _zero_i32(h3_v, NBINS3)
    _zero_f32(e3_v, NBINS3)

    @plsc.parallel_loop(0, NVEC, unroll=5, carry=jnp.zeros((L,), jnp.float32))
    def p3(i, zmid):
        v = row_v[pl.ds(i * L, L)]
        s = _ordmap_f(v)
        b1 = lax.shift_right_logical(_ub(s), SHIFT1)
        b2 = lax.shift_right_logical(_ub(s), 10) & jnp.int32(NBINS2 - 1)
        inb1 = b1 == B1
        e = jnp.exp(v - tlo)
        zmid = zmid + jnp.where(inb1 & (b2 > B2), e, jnp.float32(0.0))
        m = inb1 & (b2 == B2)
        b3 = _ub(s) & jnp.int32(NBINS3 - 1)
        plsc.addupdate_scatter(h3_v, [b3], ones, mask=m)
        plsc.addupdate_scatter(e3_v, [b3], e, mask=m)
        return zmid

    zmid = p3

    k3 = k2 - gt2
    B3, _ = _scan_hist(h3_v, NBINS3, k3)

    def tailp(i, carry):
        csum, zsum = carry
        h3 = h3_v[pl.ds(i * L, L)]
        e3 = e3_v[pl.ds(i * L, L)]
        m = (i * L + iota) >= B3
        csum = csum + jnp.where(m, h3, 0)
        zsum = zsum + jnp.where(m, e3, jnp.float32(0.0))
        return csum, zsum

    csum, zsum = lax.fori_loop(
        0, NBINS3 // L, tailp,
        (jnp.zeros((L,), jnp.int32), jnp.zeros((L,), jnp.float32)))

    t_ub = lax.shift_left(B1, SHIFT1) | lax.shift_left(B2, 10) | B3
    t_s = t_ub ^ _MIN32
    t_f = _unmap_f(t_s)
    cge = gt1 + gt2 + jnp.sum(csum)
    corr = jnp.sum(jnp.where(iota == 0,
                             jnp.exp(jnp.full((L,), t_f - tlo, jnp.float32)),
                             jnp.float32(0.0)))
    z = acc_hi + jnp.sum(zmid) + jnp.sum(zsum) \
        - (cge - jnp.int32(K_BIG)).astype(jnp.float32) * corr

    if collect40:
        n40 = jnp.minimum(_scalar(b40), CAP40)
        return t_s, tlo, z, n40
    return t_s, tlo, z, jnp.int32(0)


def _sc_body(exp_hbm, ama_hbm, out_hbm, row_v, h1_v, h2_v, h3_v, e3_v,
             c40v_v, c40i_v, sel_s_v, sel_i_v, score_v):
    iota = lax.iota(jnp.int32, L)

    def row_body(r, carry):
        row = wid = lax.axis_index("s") * NC + lax.axis_index("c")
        row = wid * ROWS_PER_W + r

        pltpu.sync_copy(exp_hbm.at[row], row_v)
        te_s, tlo_e, z_e, n40 = _process_tensor(
            row_v, h1_v, h2_v, h3_v, e3_v, True, c40v_v, c40i_v)

        T40 = _kth_largest_arr(c40v_v, n40, jnp.int32(K_SEL))
        cnt_gt = _count_ge_arr(c40v_v, n40, T40 + 1)
        need = jnp.int32(K_SEL) - cnt_gt
        nv40 = (n40 + L - 1) // L

        for j in range(3):
            sel_s_v[pl.ds(j * L, L)] = jnp.zeros((L,), jnp.int32)
            sel_i_v[pl.ds(j * L, L)] = jnp.zeros((L,), jnp.int32)

        def selbody(i, carry):
            kb, tb = carry
            s = c40v_v[pl.ds(i * L, L)]
            ix = c40i_v[pl.ds(i * L, L)]
            valid = (i * L + iota) < n40
            mgt = valid & (s > T40)
            mtie = valid & (s == T40)
            trank = tb + plsc.cumsum(mtie.astype(jnp.int32)) - 1
            keep = mgt | (mtie & (trank < need))
            offs = kb + plsc.cumsum(keep.astype(jnp.int32)) - 1
            okm = keep & (offs < jnp.int32(3 * L))
            plsc.store_scatter(sel_s_v, [offs], s, mask=okm)
            plsc.store_scatter(sel_i_v, [offs], ix, mask=okm)
            kb = kb + plsc.all_reduce_population_count(keep)
            tb = tb + plsc.all_reduce_population_count(mtie)
            return kb, tb

        z0 = jnp.zeros((L,), jnp.int32)
        lax.fori_loop(0, nv40, selbody, (z0, z0))

        pltpu.sync_copy(ama_hbm.at[row], row_v)
        ta_s, tlo_a, z_a, _ = _process_tensor(
            row_v, h1_v, h2_v, h3_v, e3_v, False, c40v_v, c40i_v)

        for j in range(3):
            s40 = sel_s_v[pl.ds(j * L, L)]
            i40 = sel_i_v[pl.ds(j * L, L)]
            v40 = _unmap_f(s40)
            pe = jnp.exp(v40 - tlo_e) / z_e
            la = plsc.load_gather(row_v, [i40])
            sa = _ordmap_f(la)
            pa = jnp.where(sa >= ta_s, jnp.exp(la - tlo_a),
                           jnp.float32(0.0)) / z_a
            ratio = pe / (pa + jnp.float32(1e-8))
            score_v[pl.ds(j * L, L)] = _poly_ln(ratio)

        ninf = jnp.full((L,), -jnp.inf, jnp.float32)

        @plsc.parallel_loop(0, NVEC, unroll=10)
        def memset(i):
            row_v[pl.ds(i * L, L)] = ninf
        for j in range(3):
            i40 = sel_i_v[pl.ds(j * L, L)]
            sc = score_v[pl.ds(j * L, L)]
            slot = j * L + iota
            plsc.store_scatter(row_v, [i40], sc, mask=slot < jnp.int32(K_SEL))
        pltpu.sync_copy(row_v, out_hbm.at[row])
        return carry

    lax.fori_loop(0, ROWS_PER_W, row_body, 0)


def _make_sc_kernel():
    mesh = plsc.VectorSubcoreMesh(core_axis_name="c", subcore_axis_name="s")
    return pl.kernel(
        _sc_body,
        out_type=[jax.ShapeDtypeStruct((N_ROWS, V), jnp.float32)],
        mesh=mesh,
        scratch_types=[
            pltpu.VMEM((V,), jnp.float32),
            pltpu.VMEM((NBINS1,), jnp.int32),
            pltpu.VMEM((NBINS2,), jnp.int32),
            pltpu.VMEM((NBINS3,), jnp.int32),
            pltpu.VMEM((NBINS3,), jnp.float32),
            pltpu.VMEM((CAP40,), jnp.int32),
            pltpu.VMEM((CAP40,), jnp.int32),
            pltpu.VMEM((3 * L,), jnp.int32),
            pltpu.VMEM((3 * L,), jnp.int32),
            pltpu.VMEM((3 * L,), jnp.float32),
        ],
        compiler_params=pltpu.CompilerParams(needs_layout_passes=False),
    )


def kernel(logits_exp, logits_ama):
    (out,) = _make_sc_kernel()(logits_exp, logits_ama)
    return out
